# baseline (speedup 1.0000x reference)
import functools

import jax
import jax.numpy as jnp
from jax import lax
from jax.experimental import pallas as pl
from jax.experimental.pallas import tpu as pltpu
from jax.experimental.pallas import tpu_sc as plsc

B, L, D = 16, 4096, 1024
NC, NS = 2, 16
NW = NC * NS
W = D // NW
R = 512


def _body(t_hbm, len_hbm, out_hbm, len_v, buf, out_v):
    wid = lax.axis_index("s") * NC + lax.axis_index("c")
    d0 = wid * W
    pltpu.sync_copy(len_hbm, len_v.at[pl.ds(0, B)])

    def batch_body(b, _):
        len_b = len_v[pl.ds(b, 16)][0]
        nchunks = (len_b + (R - 1)) // R

        def chunk_body(c, carry):
            acc0, acc1 = carry
            l0 = c * R
            pltpu.sync_copy(t_hbm.at[b, pl.ds(l0, R), pl.ds(d0, W)], buf)
            nvalid = jnp.minimum(R, len_b - l0)

            def row_body(r, carry2):
                a0, a1 = carry2
                a0 = a0 + buf[r, pl.ds(0, 16)]
                a1 = a1 + buf[r, pl.ds(16, 16)]
                return (a0, a1)

            return lax.fori_loop(0, nvalid, row_body, (acc0, acc1))

        zero = jnp.zeros((16,), jnp.float32)
        acc0, acc1 = lax.fori_loop(0, nchunks, chunk_body, (zero, zero))
        denom = jnp.broadcast_to(len_b.astype(jnp.float32), (16,))
        out_v[b, pl.ds(0, 16)] = acc0 / denom
        out_v[b, pl.ds(16, 16)] = acc1 / denom
        return 0

    lax.fori_loop(0, B, batch_body, 0)
    pltpu.sync_copy(out_v, out_hbm.at[:, pl.ds(d0, W)])


@jax.jit
def _pooled(tensor, lengths):
    mesh = plsc.VectorSubcoreMesh(
        core_axis_name="c", subcore_axis_name="s",
        num_cores=NC, num_subcores=NS)
    f = pl.kernel(
        _body,
        out_type=jax.ShapeDtypeStruct((B, D), jnp.float32),
        mesh=mesh,
        compiler_params=pltpu.CompilerParams(use_tc_tiling_on_sc=False),
        scratch_types=[
            pltpu.VMEM((B + 16,), jnp.int32),
            pltpu.VMEM((R, W), jnp.float32),
            pltpu.VMEM((B, W), jnp.float32),
        ],
    )
    return f(tensor, lengths)


def kernel(tensor, lengths):
    return _pooled(tensor, lengths.astype(jnp.int32))


# speedup vs baseline: 1.2493x; 1.2493x over previous
import functools

import jax
import jax.numpy as jnp
from jax import lax
from jax.experimental import pallas as pl
from jax.experimental.pallas import tpu as pltpu
from jax.experimental.pallas import tpu_sc as plsc

B, L, D = 16, 4096, 1024
NC, NS = 2, 16
NW = NC * NS
W = D // NW
R = 256


def _body(t_hbm, len_hbm, out_hbm, len_v, buf, out_v, sems):
    wid = lax.axis_index("s") * NC + lax.axis_index("c")
    d0 = wid * W
    pltpu.sync_copy(len_hbm, len_v.at[pl.ds(0, B)])

    def start(b, c, par):
        pltpu.async_copy(
            t_hbm.at[b, pl.ds(c * R, R), pl.ds(d0, W)], buf.at[par],
            sems.at[par])

    def wait(b, c, par):
        pltpu.make_async_copy(
            t_hbm.at[b, pl.ds(c * R, R), pl.ds(d0, W)], buf.at[par],
            sems.at[par]).wait()

    def batch_body(b, _):
        len_b = len_v[pl.ds(b, 16)][0]
        nchunks = (len_b + (R - 1)) // R
        start(b, 0, lax.rem(b, 2))

        def chunk_body(c, carry):
            acc0, acc1 = carry
            par = lax.rem(b + c, 2)
            wait(b, c, par)

            @pl.when(c + 1 < nchunks)
            def _():
                start(b, c + 1, 1 - par)

            rows_valid = len_b - c * R

            def row_masked(r, carry2):
                a0, a1 = carry2
                m = jnp.broadcast_to(
                    (r < rows_valid).astype(jnp.float32), (16,))
                a0 = a0 + buf[par, r, pl.ds(0, 16)] * m
                a1 = a1 + buf[par, r, pl.ds(16, 16)] * m
                return (a0, a1)

            return lax.fori_loop(0, R, row_masked, (acc0, acc1), unroll=8)

        zero = jnp.zeros((16,), jnp.float32)
        acc0, acc1 = lax.fori_loop(0, nchunks, chunk_body, (zero, zero))
        denom = jnp.broadcast_to(len_b.astype(jnp.float32), (16,))
        out_v[b, pl.ds(0, 16)] = acc0 / denom
        out_v[b, pl.ds(16, 16)] = acc1 / denom
        return 0

    lax.fori_loop(0, B, batch_body, 0)
    pltpu.sync_copy(out_v, out_hbm.at[:, pl.ds(d0, W)])


@jax.jit
def _pooled(tensor, lengths):
    mesh = plsc.VectorSubcoreMesh(
        core_axis_name="c", subcore_axis_name="s",
        num_cores=NC, num_subcores=NS)
    f = pl.kernel(
        _body,
        out_type=jax.ShapeDtypeStruct((B, D), jnp.float32),
        mesh=mesh,
        compiler_params=pltpu.CompilerParams(use_tc_tiling_on_sc=False),
        scratch_types=[
            pltpu.VMEM((B + 16,), jnp.int32),
            pltpu.VMEM((2, R, W), jnp.float32),
            pltpu.VMEM((B, W), jnp.float32),
            pltpu.SemaphoreType.DMA((2,)),
        ],
    )
    return f(tensor, lengths)


def kernel(tensor, lengths):
    return _pooled(tensor, lengths.astype(jnp.int32))


# speedup vs baseline: 2.9430x; 2.3557x over previous
import functools

import jax
import jax.numpy as jnp
from jax import lax
from jax.experimental import pallas as pl
from jax.experimental.pallas import tpu as pltpu
from jax.experimental.pallas import tpu_sc as plsc

B, L, D = 16, 4096, 1024
NC, NS = 2, 16
G = 4
DCW = 128
NV = DCW // 16
R = 64


def _body(t_hbm, len_hbm, out_hbm, len_v, buf, accv, tmpv, shared, sems):
    cid = lax.axis_index("c")
    sid = lax.axis_index("s")
    g = lax.rem(sid, G)
    d0 = pl.multiple_of((cid * NS + sid - g) // G * DCW, DCW)
    pltpu.sync_copy(len_hbm, len_v.at[pl.ds(0, B)])

    def start(b, c, par):
        pltpu.async_copy(
            t_hbm.at[b, pl.ds(pl.multiple_of(c * R, R), R), pl.ds(d0, DCW)],
            buf.at[par], sems.at[par])

    def wait(b, c, par):
        pltpu.make_async_copy(
            t_hbm.at[b, pl.ds(pl.multiple_of(c * R, R), R), pl.ds(d0, DCW)],
            buf.at[par], sems.at[par]).wait()

    zero = jnp.zeros((16,), jnp.float32)

    def batch_body(b, _):
        len_b = len_v[pl.ds(b, 16)][0]
        nchunks = (len_b + (R - 1)) // R
        nk = lax.max(0, (nchunks - g + (G - 1)) // G)
        start(b, g, lax.rem(b, 2))

        def chunk_body(k, carry):
            c = g + k * G
            par = lax.rem(b + k, 2)
            wait(b, c, par)

            @pl.when(k + 1 < nk)
            def _():
                start(b, c + G, 1 - par)

            rows_valid = len_b - c * R

            def row_body(r, acc):
                m = jnp.broadcast_to(
                    (r < rows_valid).astype(jnp.float32), (16,))
                return tuple(
                    acc[x] + buf[par, r, pl.ds(16 * x, 16)] * m
                    for x in range(NV))

            return lax.fori_loop(0, R, row_body, carry, unroll=4)

        acc = lax.fori_loop(0, nk, chunk_body, (zero,) * NV)
        for x in range(NV):
            accv[b, pl.ds(16 * x, 16)] = acc[x]
        return 0

    lax.fori_loop(0, B, batch_body, 0)

    pltpu.sync_copy(accv, shared.at[sid])
    plsc.subcore_barrier()

    @pl.when(g == 0)
    def _():
        for j in range(1, G):
            pltpu.sync_copy(shared.at[sid + j], tmpv)

            def add_body(i, _):
                r = i // NV
                x = lax.rem(i, NV)
                o = pl.ds(16 * x, 16)
                accv[r, o] = accv[r, o] + tmpv[r, o]
                return 0

            lax.fori_loop(0, B * NV, add_body, 0, unroll=4)

        def div_body(b, _):
            len_b = len_v[pl.ds(b, 16)][0]
            denom = jnp.broadcast_to(len_b.astype(jnp.float32), (16,))
            for x in range(NV):
                accv[b, pl.ds(16 * x, 16)] = accv[b, pl.ds(16 * x, 16)] / denom
            return 0

        lax.fori_loop(0, B, div_body, 0)
        pltpu.sync_copy(accv, out_hbm.at[:, pl.ds(d0, DCW)])


@jax.jit
def _pooled(tensor, lengths):
    mesh = plsc.VectorSubcoreMesh(
        core_axis_name="c", subcore_axis_name="s",
        num_cores=NC, num_subcores=NS)
    f = pl.kernel(
        _body,
        out_type=jax.ShapeDtypeStruct((B, D), jnp.float32),
        mesh=mesh,
        scratch_types=[
            pltpu.VMEM((B + 16,), jnp.int32),
            pltpu.VMEM((2, R, DCW), jnp.float32),
            pltpu.VMEM((B, DCW), jnp.float32),
            pltpu.VMEM((B, DCW), jnp.float32),
            pltpu.VMEM_SHARED((NS, B, DCW), jnp.float32),
            pltpu.SemaphoreType.DMA((2,)),
        ],
    )
    return f(tensor, lengths)


def kernel(tensor, lengths):
    return _pooled(tensor, lengths.astype(jnp.int32))


# speedup vs baseline: 5.7620x; 1.9579x over previous
import functools

import jax
import jax.numpy as jnp
from jax import lax
from jax.experimental import pallas as pl
from jax.experimental.pallas import tpu as pltpu
from jax.experimental.pallas import tpu_sc as plsc

B, L, D = 16, 4096, 1024
NC, NS = 2, 16
G = 4
DCW = 128
NV = DCW // 16
R = 64
PD = 4
MAXC = B * (L // (R * G))


def _body(t_hbm, len_hbm, out_hbm, len_v, buf, accv, tmpv, shared,
          sb, sl0, srv, slast, sems):
    cid = lax.axis_index("c")
    sid = lax.axis_index("s")
    g = lax.rem(sid, G)
    d0 = pl.multiple_of((cid * NS + sid - g) // G * DCW, DCW)
    pltpu.sync_copy(len_hbm, len_v.at[pl.ds(0, B)])

    def build_batch(b, j):
        len_b = len_v[pl.ds(b, 16)][0]
        nchunks = (len_b + (R - 1)) // R
        nk = lax.max(0, (nchunks - g + (G - 1)) // G)

        def put(k, j2):
            c = g + k * G
            sb[j2] = b
            sl0[j2] = c * R
            srv[j2] = len_b - c * R
            slast[j2] = (k == nk - 1).astype(jnp.int32)
            return j2 + 1

        return lax.fori_loop(0, nk, put, j)

    t_total = lax.fori_loop(0, B, build_batch, 0)

    def zero_body(i, _):
        accv[i // NV, pl.ds(16 * lax.rem(i, NV), 16)] = jnp.zeros(
            (16,), jnp.float32)
        return 0

    lax.fori_loop(0, B * NV, zero_body, 0, unroll=4)

    def start(j, par):
        l0 = pl.multiple_of(sl0[j], R)
        pltpu.async_copy(
            t_hbm.at[sb[j], pl.ds(l0, R), pl.ds(d0, DCW)],
            buf.at[par], sems.at[par])

    def wait(j, par):
        l0 = pl.multiple_of(sl0[j], R)
        pltpu.make_async_copy(
            t_hbm.at[sb[j], pl.ds(l0, R), pl.ds(d0, DCW)],
            buf.at[par], sems.at[par]).wait()

    for i in range(PD - 1):
        @pl.when(i < t_total)
        def _():
            start(i, i)

    zero = jnp.zeros((16,), jnp.float32)

    def flat_body(j, acc):
        par = lax.rem(j, PD)
        wait(j, par)

        @pl.when(j + (PD - 1) < t_total)
        def _():
            start(j + (PD - 1), lax.rem(j + (PD - 1), PD))

        rows_valid = srv[j]

        def row_body(r, a):
            m = jnp.broadcast_to(
                (r < rows_valid).astype(jnp.float32), (16,))
            return tuple(
                a[x] + buf[par, r, pl.ds(16 * x, 16)] * m
                for x in range(NV))

        acc = lax.fori_loop(0, R, row_body, acc, unroll=4)
        b = sb[j]
        last = slast[j]

        @pl.when(last == 1)
        def _():
            for x in range(NV):
                accv[b, pl.ds(16 * x, 16)] = acc[x]

        keep = jnp.broadcast_to((last == 0).astype(jnp.float32), (16,))
        return tuple(a * keep for a in acc)

    lax.fori_loop(0, t_total, flat_body, (zero,) * NV)

    pltpu.sync_copy(accv, shared.at[sid])
    plsc.subcore_barrier()

    @pl.when(g == 0)
    def _():
        for j in range(1, G):
            pltpu.sync_copy(shared.at[sid + j], tmpv)

            def add_body(i, _):
                o = pl.ds(16 * lax.rem(i, NV), 16)
                r = i // NV
                accv[r, o] = accv[r, o] + tmpv[r, o]
                return 0

            lax.fori_loop(0, B * NV, add_body, 0, unroll=4)

        def div_body(b, _):
            len_b = len_v[pl.ds(b, 16)][0]
            denom = jnp.broadcast_to(len_b.astype(jnp.float32), (16,))
            for x in range(NV):
                accv[b, pl.ds(16 * x, 16)] = accv[b, pl.ds(16 * x, 16)] / denom
            return 0

        lax.fori_loop(0, B, div_body, 0)
        pltpu.sync_copy(accv, out_hbm.at[:, pl.ds(d0, DCW)])


@jax.jit
def _pooled(tensor, lengths):
    mesh = plsc.VectorSubcoreMesh(
        core_axis_name="c", subcore_axis_name="s",
        num_cores=NC, num_subcores=NS)
    f = pl.kernel(
        _body,
        out_type=jax.ShapeDtypeStruct((B, D), jnp.float32),
        mesh=mesh,
        scratch_types=[
            pltpu.VMEM((B + 16,), jnp.int32),
            pltpu.VMEM((PD, R, DCW), jnp.float32),
            pltpu.VMEM((B, DCW), jnp.float32),
            pltpu.VMEM((B, DCW), jnp.float32),
            pltpu.VMEM_SHARED((NS, B, DCW), jnp.float32),
            pltpu.SMEM((MAXC,), jnp.int32),
            pltpu.SMEM((MAXC,), jnp.int32),
            pltpu.SMEM((MAXC,), jnp.int32),
            pltpu.SMEM((MAXC,), jnp.int32),
            pltpu.SemaphoreType.DMA((PD,)),
        ],
    )
    return f(tensor, lengths)


def kernel(tensor, lengths):
    return _pooled(tensor, lengths.astype(jnp.int32))
